# baseline (speedup 1.0000x reference)
import jax
import jax.numpy as jnp
from jax.experimental import pallas as pl

_N, _K, _D = 4096, 512, 64
_TN = 256
_KC = 128


def _assign_kernel(x_ref, c_ref, out_ref):
    xt = x_ref[...]
    best_d = jnp.full((_TN, 1), jnp.inf, jnp.float32)
    best_i = jnp.zeros((_TN, 1), jnp.int32)
    for k0 in range(0, _K, _KC):
        c = c_ref[k0:k0 + _KC, :]
        cn = jnp.sum(c * c, axis=1)[None, :]
        scores = jax.lax.dot_general(
            xt, c,
            dimension_numbers=(((1,), (1,)), ((), ())),
            preferred_element_type=jnp.float32,
            precision=jax.lax.Precision.HIGHEST,
        )
        dist = cn - 2.0 * scores
        d = jnp.min(dist, axis=1, keepdims=True)
        i = jnp.argmin(dist, axis=1)[:, None].astype(jnp.int32) + k0
        take = d < best_d
        best_d = jnp.where(take, d, best_d)
        best_i = jnp.where(take, i, best_i)
    out_ref[...] = best_i[:, 0]


def kernel(x, centers):
    return pl.pallas_call(
        _assign_kernel,
        grid=(_N // _TN,),
        in_specs=[
            pl.BlockSpec((_TN, _D), lambda i: (i, 0)),
            pl.BlockSpec((_K, _D), lambda i: (0, 0)),
        ],
        out_specs=pl.BlockSpec((_TN,), lambda i: (i,)),
        out_shape=jax.ShapeDtypeStruct((_N,), jnp.int32),
    )(x, centers)


# speedup vs baseline: 25.5041x; 25.5041x over previous
import jax
import jax.numpy as jnp
from jax.experimental import pallas as pl

_N, _K, _D = 4096, 512, 64
_TN = 128


def _assign_kernel(xt_ref, c_ref, out_ref):
    c = c_ref[...]
    cn = jnp.sum(c * c, axis=1)[:, None]
    scores = jax.lax.dot_general(
        c, xt_ref[...],
        dimension_numbers=(((1,), (0,)), ((), ())),
        preferred_element_type=jnp.float32,
        precision=jax.lax.Precision.HIGHEST,
    )
    dist = cn - 2.0 * scores
    out_ref[...] = jnp.argmin(dist, axis=0).astype(jnp.int32)


def kernel(x, centers):
    xt = x.T
    return pl.pallas_call(
        _assign_kernel,
        grid=(_N // _TN,),
        in_specs=[
            pl.BlockSpec((_D, _TN), lambda i: (0, i)),
            pl.BlockSpec((_K, _D), lambda i: (0, 0)),
        ],
        out_specs=pl.BlockSpec((_TN,), lambda i: (i,)),
        out_shape=jax.ShapeDtypeStruct((_N,), jnp.int32),
    )(xt, centers)
